# baseline (speedup 1.0000x reference)
import functools

import jax
import jax.numpy as jnp
import numpy as np
from jax import lax
from jax.experimental import pallas as pl
from jax.experimental.pallas import tpu as pltpu
from jax.experimental.pallas import tpu_sc as plsc

VOCAB = 100000
DIM = 128
B = 16384
L = 200
HID = 50
OUT = 4

NC = 2
NS = 16
NW = NC * NS
SEQ_PER_W = B // NW
GROUP = 64
NGROUP = SEQ_PER_W // GROUP
PDIM = DIM // 2
PVR = PDIM // 16

_PERM = np.empty(DIM, np.int32)
for _c in range(PVR):
    for _k in range(16):
        _PERM[32 * _c + _k] = 32 * _c + 2 * _k
        _PERM[32 * _c + 16 + _k] = 32 * _c + 2 * _k + 1


def _sc_body(x_hbm, table_hbm, sums_hbm, idx_v, rows0, rows1, out_v, sem0, sem1):
    wid = lax.axis_index("s") * NC + lax.axis_index("c")
    seq0 = wid * SEQ_PER_W

    def issue(s_local, rows_ref, sem):
        base = s_local * L
        pltpu.async_copy(table_hbm.at[idx_v.at[pl.ds(base, 128)]],
                         rows_ref.at[pl.ds(0, 128)], sem)
        pltpu.async_copy(table_hbm.at[idx_v.at[pl.ds(base + 128, L - 128)]],
                         rows_ref.at[pl.ds(128, L - 128)], sem)

    def drain(rows_ref, sem):
        pltpu.make_async_copy(table_hbm.at[pl.ds(0, L)], rows_ref, sem).wait()

    def reduce(rows_ref, s_local):
        def add_row(r, acc):
            out = list(acc)
            for c in range(PVR):
                v = rows_ref[r, pl.ds(16 * c, 16)]
                lo = plsc.bitcast(v << 16, jnp.float32)
                hi = plsc.bitcast(v & jnp.int32(-65536), jnp.float32)
                out[2 * c] = out[2 * c] + lo
                out[2 * c + 1] = out[2 * c + 1] + hi
            return out

        def body(r2, acc):
            return tuple(add_row(2 * r2 + 1, add_row(2 * r2, acc)))

        acc = lax.fori_loop(
            0, L // 2, body,
            tuple(jnp.zeros((16,), jnp.float32) for _ in range(2 * PVR)))
        for c in range(PVR):
            out_v[s_local, pl.ds(32 * c, 16)] = acc[2 * c]
            out_v[s_local, pl.ds(32 * c + 16, 16)] = acc[2 * c + 1]

    @pl.loop(0, NGROUP)
    def _group(g):
        gseq = seq0 + g * GROUP
        pltpu.sync_copy(x_hbm.at[pl.ds(gseq * L, GROUP * L)], idx_v)
        issue(0, rows0, sem0)

        @pl.loop(0, GROUP, step=2)
        def _seq(s):
            issue(s + 1, rows1, sem1)
            drain(rows0, sem0)
            reduce(rows0, s)

            @pl.when(s + 2 < GROUP)
            def _():
                issue(s + 2, rows0, sem0)
            drain(rows1, sem1)
            reduce(rows1, s + 1)

        pltpu.sync_copy(out_v, sums_hbm.at[pl.ds(gseq, GROUP)])


_sc_sum = functools.partial(
    pl.kernel,
    out_type=jax.ShapeDtypeStruct((B, DIM), jnp.float32),
    mesh=plsc.VectorSubcoreMesh(core_axis_name="c", subcore_axis_name="s"),
    compiler_params=pltpu.CompilerParams(
        needs_layout_passes=False, use_tc_tiling_on_sc=False),
    scratch_types=[
        pltpu.VMEM((GROUP * L,), jnp.int32),
        pltpu.VMEM((L, PDIM), jnp.int32),
        pltpu.VMEM((L, PDIM), jnp.int32),
        pltpu.VMEM((GROUP, DIM), jnp.float32),
        pltpu.SemaphoreType.DMA,
        pltpu.SemaphoreType.DMA,
    ],
)(_sc_body)


BLK = 2048


def _mlp_body(x_ref, sums_ref, w1_ref, b1_ref, w2_ref, b2_ref, out_ref):
    xb = x_ref[...]
    lengths = jnp.sum((xb != 0).astype(jnp.float32), axis=1, keepdims=True)
    pooled = sums_ref[...] / lengths
    h = jnp.dot(pooled, w1_ref[...], preferred_element_type=jnp.float32)
    h = jnp.maximum(h + b1_ref[...], 0.0)
    out_ref[...] = (jnp.dot(h, w2_ref[...], preferred_element_type=jnp.float32)
                    + b2_ref[...])


def _mlp(x2d, sums, w1, b1, w2, b2):
    return pl.pallas_call(
        _mlp_body,
        grid=(B // BLK,),
        in_specs=[
            pl.BlockSpec((BLK, L), lambda i: (i, 0)),
            pl.BlockSpec((BLK, DIM), lambda i: (i, 0)),
            pl.BlockSpec((DIM, HID), lambda i: (0, 0)),
            pl.BlockSpec((1, HID), lambda i: (0, 0)),
            pl.BlockSpec((HID, OUT), lambda i: (0, 0)),
            pl.BlockSpec((1, OUT), lambda i: (0, 0)),
        ],
        out_specs=pl.BlockSpec((BLK, OUT), lambda i: (i, 0)),
        out_shape=jax.ShapeDtypeStruct((B, OUT), jnp.float32),
    )(x2d, sums, w1, b1.reshape(1, HID), w2, b2.reshape(1, OUT))


def kernel(x, table, W1, b1, W2, b2):
    x32 = x.astype(jnp.int32)
    packed = jax.lax.bitcast_convert_type(
        table.astype(jnp.bfloat16).reshape(VOCAB, PDIM, 2), jnp.int32)
    sums = _sc_sum(x32.reshape(B * L), packed)
    return _mlp(x32, sums, W1[_PERM, :], b1, W2, b2)


# speedup vs baseline: 1.5156x; 1.5156x over previous
import functools

import jax
import jax.numpy as jnp
import numpy as np
from jax import lax
from jax.experimental import pallas as pl
from jax.experimental.pallas import tpu as pltpu
from jax.experimental.pallas import tpu_sc as plsc

VOCAB = 100000
DIM = 128
B = 16384
L = 200
HID = 50
OUT = 4

NC = 2
NS = 16
NW = NC * NS
SEQ_PER_W = B // NW
GROUP = 64
NGROUP = SEQ_PER_W // GROUP
PDIM = DIM // 2
PVR = PDIM // 16

_PERM = np.empty(DIM, np.int32)
for _c in range(PVR):
    for _k in range(16):
        _PERM[32 * _c + _k] = 16 * _c + _k
        _PERM[32 * _c + 16 + _k] = 64 + 16 * _c + _k


def _pack_table(table):
    u = jax.lax.bitcast_convert_type(table, jnp.uint32)
    rb = (u + jnp.uint32(0x7FFF) + ((u >> 16) & jnp.uint32(1))) >> 16
    packed = (rb[:, PDIM:] << 16) | rb[:, :PDIM]
    return jax.lax.bitcast_convert_type(packed, jnp.int32)


def _sc_body(x_hbm, table_hbm, sums_hbm, idx_v, rows0, rows1, out_v, sem0, sem1):
    wid = lax.axis_index("s") * NC + lax.axis_index("c")
    seq0 = wid * SEQ_PER_W

    def issue(s_local, rows_ref, sem):
        base = s_local * L
        pltpu.async_copy(table_hbm.at[idx_v.at[pl.ds(base, 128)]],
                         rows_ref.at[pl.ds(0, 128)], sem)
        pltpu.async_copy(table_hbm.at[idx_v.at[pl.ds(base + 128, L - 128)]],
                         rows_ref.at[pl.ds(128, L - 128)], sem)

    def drain(rows_ref, sem):
        pltpu.make_async_copy(table_hbm.at[pl.ds(0, L)], rows_ref, sem).wait()

    def reduce(rows_ref, s_local):
        def add_row(r, acc):
            out = list(acc)
            for c in range(PVR):
                v = rows_ref[r, pl.ds(16 * c, 16)]
                lo = plsc.bitcast(v << 16, jnp.float32)
                hi = plsc.bitcast(v & jnp.int32(-65536), jnp.float32)
                out[2 * c] = out[2 * c] + lo
                out[2 * c + 1] = out[2 * c + 1] + hi
            return out

        def body(r2, acc):
            return tuple(add_row(2 * r2 + 1, add_row(2 * r2, acc)))

        acc = lax.fori_loop(
            0, L // 2, body,
            tuple(jnp.zeros((16,), jnp.float32) for _ in range(2 * PVR)))
        for c in range(PVR):
            out_v[s_local, pl.ds(32 * c, 16)] = acc[2 * c]
            out_v[s_local, pl.ds(32 * c + 16, 16)] = acc[2 * c + 1]

    @pl.loop(0, NGROUP)
    def _group(g):
        gseq = seq0 + g * GROUP
        pltpu.sync_copy(x_hbm.at[pl.ds(gseq * L, GROUP * L)], idx_v)
        issue(0, rows0, sem0)

        @pl.loop(0, GROUP, step=2)
        def _seq(s):
            issue(s + 1, rows1, sem1)
            drain(rows0, sem0)
            reduce(rows0, s)

            @pl.when(s + 2 < GROUP)
            def _():
                issue(s + 2, rows0, sem0)
            drain(rows1, sem1)
            reduce(rows1, s + 1)

        pltpu.sync_copy(out_v, sums_hbm.at[pl.ds(gseq, GROUP)])


@functools.cache
def _sc_sum():
    return pl.kernel(
        _sc_body,
        out_type=jax.ShapeDtypeStruct((B, DIM), jnp.float32),
        mesh=plsc.VectorSubcoreMesh(
            core_axis_name="c", subcore_axis_name="s",
            num_cores=NC, num_subcores=NS),
        compiler_params=pltpu.CompilerParams(
            needs_layout_passes=False, use_tc_tiling_on_sc=False),
        scratch_types=[
            pltpu.VMEM((GROUP * L,), jnp.int32),
            pltpu.VMEM((L, PDIM), jnp.int32),
            pltpu.VMEM((L, PDIM), jnp.int32),
            pltpu.VMEM((GROUP, DIM), jnp.float32),
            pltpu.SemaphoreType.DMA,
            pltpu.SemaphoreType.DMA,
        ],
    )


BLK = 2048


def _mlp_body(x_ref, sums_ref, w1_ref, b1_ref, w2_ref, b2_ref, out_ref):
    xb = x_ref[...]
    lengths = jnp.sum((xb != 0).astype(jnp.float32), axis=1, keepdims=True)
    pooled = sums_ref[...] / lengths
    h = jnp.dot(pooled, w1_ref[...], preferred_element_type=jnp.float32)
    h = jnp.maximum(h + b1_ref[...], 0.0)
    out_ref[...] = (jnp.dot(h, w2_ref[...], preferred_element_type=jnp.float32)
                    + b2_ref[...])


def _mlp(x2d, sums, w1, b1, w2, b2):
    return pl.pallas_call(
        _mlp_body,
        grid=(B // BLK,),
        in_specs=[
            pl.BlockSpec((BLK, L), lambda i: (i, 0)),
            pl.BlockSpec((BLK, DIM), lambda i: (i, 0)),
            pl.BlockSpec((DIM, HID), lambda i: (0, 0)),
            pl.BlockSpec((1, HID), lambda i: (0, 0)),
            pl.BlockSpec((HID, OUT), lambda i: (0, 0)),
            pl.BlockSpec((1, OUT), lambda i: (0, 0)),
        ],
        out_specs=pl.BlockSpec((BLK, OUT), lambda i: (i, 0)),
        out_shape=jax.ShapeDtypeStruct((B, OUT), jnp.float32),
    )(x2d, sums, w1, b1.reshape(1, HID), w2, b2.reshape(1, OUT))


def kernel(x, table, W1, b1, W2, b2):
    x32 = x.astype(jnp.int32)
    sums = _sc_sum()(x32.reshape(B * L), _pack_table(table))
    return _mlp(x32, sums, W1[_PERM, :], b1, W2, b2)


# speedup vs baseline: 1.5409x; 1.0167x over previous
import functools

import jax
import jax.numpy as jnp
import numpy as np
from jax import lax
from jax.experimental import pallas as pl
from jax.experimental.pallas import tpu as pltpu
from jax.experimental.pallas import tpu_sc as plsc

VOCAB = 100000
DIM = 128
B = 16384
L = 200
HID = 50
OUT = 4

NC = 2
NS = 16
NW = NC * NS
SEQ_PER_W = B // NW
GROUP = 128
NGROUP = SEQ_PER_W // GROUP
PDIM = DIM // 2
PVR = PDIM // 16

_PERM = np.empty(DIM, np.int32)
for _c in range(PVR):
    for _k in range(16):
        _PERM[32 * _c + _k] = 16 * _c + _k
        _PERM[32 * _c + 16 + _k] = 64 + 16 * _c + _k


def _pack_table(table):
    u = jax.lax.bitcast_convert_type(table, jnp.uint32)
    rb = (u + jnp.uint32(0x7FFF) + ((u >> 16) & jnp.uint32(1))) >> 16
    packed = (rb[:, PDIM:] << 16) | rb[:, :PDIM]
    return jax.lax.bitcast_convert_type(packed, jnp.int32)


def _sc_body(x_hbm, table_hbm, sums_hbm, idx0, idx1, rows0, rows1, out_v,
             isem, sem0, sem1):
    wid = lax.axis_index("s") * NC + lax.axis_index("c")
    seq0 = wid * SEQ_PER_W
    idx_bufs = (idx0, idx1)

    def stage_idx(g, buf):
        pltpu.async_copy(
            x_hbm.at[pl.ds((seq0 + g * GROUP) * L, GROUP * L)], buf, isem)

    def wait_idx(buf):
        pltpu.make_async_copy(x_hbm.at[pl.ds(0, GROUP * L)], buf, isem).wait()

    def issue(idx_v, s_local, rows_ref, sem):
        base = s_local * L
        pltpu.async_copy(table_hbm.at[idx_v.at[pl.ds(base, 128)]],
                         rows_ref.at[pl.ds(0, 128)], sem)
        pltpu.async_copy(table_hbm.at[idx_v.at[pl.ds(base + 128, L - 128)]],
                         rows_ref.at[pl.ds(128, L - 128)], sem)

    def drain(rows_ref, sem):
        pltpu.make_async_copy(table_hbm.at[pl.ds(0, L)], rows_ref, sem).wait()

    def reduce(rows_ref, s_local):
        def add_row(r, acc):
            out = list(acc)
            for c in range(PVR):
                v = rows_ref[r, pl.ds(16 * c, 16)]
                lo = plsc.bitcast(v << 16, jnp.float32)
                hi = plsc.bitcast(v & jnp.int32(-65536), jnp.float32)
                out[2 * c] = out[2 * c] + lo
                out[2 * c + 1] = out[2 * c + 1] + hi
            return out

        def body(r4, acc):
            for k in range(4):
                acc = add_row(4 * r4 + k, acc)
            return tuple(acc)

        acc = lax.fori_loop(
            0, L // 4, body,
            tuple(jnp.zeros((16,), jnp.float32) for _ in range(2 * PVR)))
        for c in range(PVR):
            out_v[s_local, pl.ds(32 * c, 16)] = acc[2 * c]
            out_v[s_local, pl.ds(32 * c + 16, 16)] = acc[2 * c + 1]

    stage_idx(0, idx0)
    for g in range(NGROUP):
        idx_v = idx_bufs[g % 2]
        wait_idx(idx_v)
        issue(idx_v, 0, rows0, sem0)
        if g + 1 < NGROUP:
            stage_idx(g + 1, idx_bufs[(g + 1) % 2])

        @pl.loop(0, GROUP, step=2)
        def _seq(s, idx_v=idx_v):
            issue(idx_v, s + 1, rows1, sem1)
            drain(rows0, sem0)
            reduce(rows0, s)

            @pl.when(s + 2 < GROUP)
            def _():
                issue(idx_v, s + 2, rows0, sem0)
            drain(rows1, sem1)
            reduce(rows1, s + 1)

        pltpu.sync_copy(out_v, sums_hbm.at[pl.ds(seq0 + g * GROUP, GROUP)])


@functools.cache
def _sc_sum():
    return pl.kernel(
        _sc_body,
        out_type=jax.ShapeDtypeStruct((B, DIM), jnp.float32),
        mesh=plsc.VectorSubcoreMesh(
            core_axis_name="c", subcore_axis_name="s",
            num_cores=NC, num_subcores=NS),
        compiler_params=pltpu.CompilerParams(
            needs_layout_passes=False, use_tc_tiling_on_sc=False),
        scratch_types=[
            pltpu.VMEM((GROUP * L,), jnp.int32),
            pltpu.VMEM((GROUP * L,), jnp.int32),
            pltpu.VMEM((L, PDIM), jnp.int32),
            pltpu.VMEM((L, PDIM), jnp.int32),
            pltpu.VMEM((GROUP, DIM), jnp.float32),
            pltpu.SemaphoreType.DMA,
            pltpu.SemaphoreType.DMA,
            pltpu.SemaphoreType.DMA,
        ],
    )


BLK = 2048


def _mlp_body(x_ref, sums_ref, w1_ref, b1_ref, w2_ref, b2_ref, out_ref):
    xb = x_ref[...]
    lengths = jnp.sum((xb != 0).astype(jnp.float32), axis=1, keepdims=True)
    pooled = sums_ref[...] / lengths
    h = jnp.dot(pooled, w1_ref[...], preferred_element_type=jnp.float32)
    h = jnp.maximum(h + b1_ref[...], 0.0)
    out_ref[...] = (jnp.dot(h, w2_ref[...], preferred_element_type=jnp.float32)
                    + b2_ref[...])


def _mlp(x2d, sums, w1, b1, w2, b2):
    return pl.pallas_call(
        _mlp_body,
        grid=(B // BLK,),
        in_specs=[
            pl.BlockSpec((BLK, L), lambda i: (i, 0)),
            pl.BlockSpec((BLK, DIM), lambda i: (i, 0)),
            pl.BlockSpec((DIM, HID), lambda i: (0, 0)),
            pl.BlockSpec((1, HID), lambda i: (0, 0)),
            pl.BlockSpec((HID, OUT), lambda i: (0, 0)),
            pl.BlockSpec((1, OUT), lambda i: (0, 0)),
        ],
        out_specs=pl.BlockSpec((BLK, OUT), lambda i: (i, 0)),
        out_shape=jax.ShapeDtypeStruct((B, OUT), jnp.float32),
    )(x2d, sums, w1, b1.reshape(1, HID), w2, b2.reshape(1, OUT))


def kernel(x, table, W1, b1, W2, b2):
    x32 = x.astype(jnp.int32)
    sums = _sc_sum()(x32.reshape(B * L), _pack_table(table))
    return _mlp(x32, sums, W1[_PERM, :], b1, W2, b2)


# speedup vs baseline: 1.5863x; 1.0295x over previous
import functools

import jax
import jax.numpy as jnp
import numpy as np
from jax import lax
from jax.experimental import pallas as pl
from jax.experimental.pallas import tpu as pltpu
from jax.experimental.pallas import tpu_sc as plsc

VOCAB = 100000
DIM = 128
B = 16384
L = 200
HID = 50
OUT = 4

NC = 2
NS = 16
NW = NC * NS
SEQ_PER_W = B // NW
GROUP = 128
NGROUP = SEQ_PER_W // GROUP
PDIM = DIM // 2
PVR = PDIM // 16

_PERM = np.empty(DIM, np.int32)
for _c in range(PVR):
    for _k in range(16):
        _PERM[32 * _c + _k] = 16 * _c + _k
        _PERM[32 * _c + 16 + _k] = 64 + 16 * _c + _k


PACK_BLK = 2000


def _pack_body(t_ref, out_ref):
    u = jax.lax.bitcast_convert_type(t_ref[...], jnp.uint32)
    rb = (u + jnp.uint32(0x7FFF) + ((u >> 16) & jnp.uint32(1))) >> 16
    packed = (rb[:, PDIM:] << 16) | rb[:, :PDIM]
    out_ref[...] = jax.lax.bitcast_convert_type(packed, jnp.int32)


def _pack_table(table):
    return pl.pallas_call(
        _pack_body,
        grid=(VOCAB // PACK_BLK,),
        in_specs=[pl.BlockSpec((PACK_BLK, DIM), lambda i: (i, 0))],
        out_specs=pl.BlockSpec((PACK_BLK, PDIM), lambda i: (i, 0)),
        out_shape=jax.ShapeDtypeStruct((VOCAB, PDIM), jnp.int32),
    )(table)


def _sc_body(x_hbm, table_hbm, sums_hbm, idx0, idx1, rows0, rows1, out_v,
             isem, sem0, sem1):
    wid = lax.axis_index("s") * NC + lax.axis_index("c")
    seq0 = wid * SEQ_PER_W
    idx_bufs = (idx0, idx1)

    def stage_idx(g, buf):
        pltpu.async_copy(
            x_hbm.at[pl.ds((seq0 + g * GROUP) * L, GROUP * L)], buf, isem)

    def wait_idx(buf):
        pltpu.make_async_copy(x_hbm.at[pl.ds(0, GROUP * L)], buf, isem).wait()

    def issue(idx_v, s_local, rows_ref, sem):
        base = s_local * L
        pltpu.async_copy(table_hbm.at[idx_v.at[pl.ds(base, 128)]],
                         rows_ref.at[pl.ds(0, 128)], sem)
        pltpu.async_copy(table_hbm.at[idx_v.at[pl.ds(base + 128, L - 128)]],
                         rows_ref.at[pl.ds(128, L - 128)], sem)

    def drain(rows_ref, sem):
        pltpu.make_async_copy(table_hbm.at[pl.ds(0, L)], rows_ref, sem).wait()

    def reduce(rows_ref, s_local):
        def add_row(r, acc):
            out = list(acc)
            for c in range(PVR):
                v = rows_ref[r, pl.ds(16 * c, 16)]
                lo = plsc.bitcast(v << 16, jnp.float32)
                hi = plsc.bitcast(v & jnp.int32(-65536), jnp.float32)
                out[2 * c] = out[2 * c] + lo
                out[2 * c + 1] = out[2 * c + 1] + hi
            return out

        def body(r4, acc):
            for k in range(4):
                acc = add_row(4 * r4 + k, acc)
            return tuple(acc)

        acc = lax.fori_loop(
            0, L // 4, body,
            tuple(jnp.zeros((16,), jnp.float32) for _ in range(2 * PVR)))
        for c in range(PVR):
            out_v[s_local, pl.ds(32 * c, 16)] = acc[2 * c]
            out_v[s_local, pl.ds(32 * c + 16, 16)] = acc[2 * c + 1]

    stage_idx(0, idx0)
    for g in range(NGROUP):
        idx_v = idx_bufs[g % 2]
        wait_idx(idx_v)
        issue(idx_v, 0, rows0, sem0)
        if g + 1 < NGROUP:
            stage_idx(g + 1, idx_bufs[(g + 1) % 2])

        @pl.loop(0, GROUP, step=2)
        def _seq(s, idx_v=idx_v):
            issue(idx_v, s + 1, rows1, sem1)
            drain(rows0, sem0)
            reduce(rows0, s)

            @pl.when(s + 2 < GROUP)
            def _():
                issue(idx_v, s + 2, rows0, sem0)
            drain(rows1, sem1)
            reduce(rows1, s + 1)

        pltpu.sync_copy(out_v, sums_hbm.at[pl.ds(seq0 + g * GROUP, GROUP)])


@functools.cache
def _sc_sum():
    return pl.kernel(
        _sc_body,
        out_type=jax.ShapeDtypeStruct((B, DIM), jnp.float32),
        mesh=plsc.VectorSubcoreMesh(
            core_axis_name="c", subcore_axis_name="s",
            num_cores=NC, num_subcores=NS),
        compiler_params=pltpu.CompilerParams(
            needs_layout_passes=False, use_tc_tiling_on_sc=False),
        scratch_types=[
            pltpu.VMEM((GROUP * L,), jnp.int32),
            pltpu.VMEM((GROUP * L,), jnp.int32),
            pltpu.VMEM((L, PDIM), jnp.int32),
            pltpu.VMEM((L, PDIM), jnp.int32),
            pltpu.VMEM((GROUP, DIM), jnp.float32),
            pltpu.SemaphoreType.DMA,
            pltpu.SemaphoreType.DMA,
            pltpu.SemaphoreType.DMA,
        ],
    )


BLK = 2048


def _mlp_body(x_ref, sums_ref, w1_ref, b1_ref, w2_ref, b2_ref, out_ref):
    xb = x_ref[...]
    lengths = jnp.sum((xb != 0).astype(jnp.float32), axis=1, keepdims=True)
    pooled = sums_ref[...] / lengths
    h = jnp.dot(pooled, w1_ref[...], preferred_element_type=jnp.float32)
    h = jnp.maximum(h + b1_ref[...], 0.0)
    out_ref[...] = (jnp.dot(h, w2_ref[...], preferred_element_type=jnp.float32)
                    + b2_ref[...])


def _mlp(x2d, sums, w1, b1, w2, b2):
    return pl.pallas_call(
        _mlp_body,
        grid=(B // BLK,),
        in_specs=[
            pl.BlockSpec((BLK, L), lambda i: (i, 0)),
            pl.BlockSpec((BLK, DIM), lambda i: (i, 0)),
            pl.BlockSpec((DIM, HID), lambda i: (0, 0)),
            pl.BlockSpec((1, HID), lambda i: (0, 0)),
            pl.BlockSpec((HID, OUT), lambda i: (0, 0)),
            pl.BlockSpec((1, OUT), lambda i: (0, 0)),
        ],
        out_specs=pl.BlockSpec((BLK, OUT), lambda i: (i, 0)),
        out_shape=jax.ShapeDtypeStruct((B, OUT), jnp.float32),
    )(x2d, sums, w1, b1.reshape(1, HID), w2, b2.reshape(1, OUT))


def kernel(x, table, W1, b1, W2, b2):
    x32 = x.astype(jnp.int32)
    sums = _sc_sum()(x32.reshape(B * L), _pack_table(table))
    return _mlp(x32, sums, W1[_PERM, :], b1, W2, b2)


# speedup vs baseline: 2.0166x; 1.2712x over previous
import functools

import jax
import jax.numpy as jnp
import numpy as np
from jax import lax
from jax.experimental import pallas as pl
from jax.experimental.pallas import tpu as pltpu
from jax.experimental.pallas import tpu_sc as plsc

VOCAB = 100000
DIM = 128
B = 16384
L = 200
HID = 50
OUT = 4

NC = 2
NS = 16
NW = NC * NS
SEQ_PER_W = B // NW
GROUP = 64
NGROUP = SEQ_PER_W // GROUP
NSLOT = 4
PDIM = DIM // 2
PVR = PDIM // 16

_PERM = np.empty(DIM, np.int32)
for _c in range(PVR):
    for _k in range(16):
        _PERM[32 * _c + _k] = 16 * _c + _k
        _PERM[32 * _c + 16 + _k] = 64 + 16 * _c + _k


PACK_BLK = 2000


def _pack_body(t_ref, out_ref):
    u = jax.lax.bitcast_convert_type(t_ref[...], jnp.uint32)
    rb = (u + jnp.uint32(0x7FFF) + ((u >> 16) & jnp.uint32(1))) >> 16
    packed = (rb[:, PDIM:] << 16) | rb[:, :PDIM]
    out_ref[...] = jax.lax.bitcast_convert_type(packed, jnp.int32)


def _pack_table(table):
    return pl.pallas_call(
        _pack_body,
        grid=(VOCAB // PACK_BLK,),
        in_specs=[pl.BlockSpec((PACK_BLK, DIM), lambda i: (i, 0))],
        out_specs=pl.BlockSpec((PACK_BLK, PDIM), lambda i: (i, 0)),
        out_shape=jax.ShapeDtypeStruct((VOCAB, PDIM), jnp.int32),
    )(table)


def _sc_body(x_hbm, table_hbm, sums_hbm, idx0, idx1,
             rows0, rows1, rows2, rows3, out_v,
             isem, sem0, sem1, sem2, sem3):
    wid = lax.axis_index("s") * NC + lax.axis_index("c")
    seq0 = wid * SEQ_PER_W
    idx_bufs = (idx0, idx1)
    rows = (rows0, rows1, rows2, rows3)
    sems = (sem0, sem1, sem2, sem3)

    def stage_idx(g, buf):
        pltpu.async_copy(
            x_hbm.at[pl.ds((seq0 + g * GROUP) * L, GROUP * L)], buf, isem)

    def wait_idx(buf):
        pltpu.make_async_copy(x_hbm.at[pl.ds(0, GROUP * L)], buf, isem).wait()

    def issue(idx_v, s_local, k):
        base = s_local * L
        pltpu.async_copy(table_hbm.at[idx_v.at[pl.ds(base, 128)]],
                         rows[k].at[pl.ds(0, 128)], sems[k])
        pltpu.async_copy(table_hbm.at[idx_v.at[pl.ds(base + 128, L - 128)]],
                         rows[k].at[pl.ds(128, L - 128)], sems[k])

    def drain(k):
        pltpu.make_async_copy(table_hbm.at[pl.ds(0, L)], rows[k], sems[k]).wait()

    def reduce(k, s_local):
        rows_ref = rows[k]

        def add_row(r, acc):
            out = list(acc)
            for c in range(PVR):
                v = rows_ref[r, pl.ds(16 * c, 16)]
                lo = plsc.bitcast(v << 16, jnp.float32)
                hi = plsc.bitcast(v & jnp.int32(-65536), jnp.float32)
                out[2 * c] = out[2 * c] + lo
                out[2 * c + 1] = out[2 * c + 1] + hi
            return out

        def body(r4, acc):
            for j in range(4):
                acc = add_row(4 * r4 + j, acc)
            return tuple(acc)

        acc = lax.fori_loop(
            0, L // 4, body,
            tuple(jnp.zeros((16,), jnp.float32) for _ in range(2 * PVR)))
        for c in range(PVR):
            out_v[s_local, pl.ds(32 * c, 16)] = acc[2 * c]
            out_v[s_local, pl.ds(32 * c + 16, 16)] = acc[2 * c + 1]

    stage_idx(0, idx0)
    wait_idx(idx0)
    for k in range(NSLOT - 1):
        issue(idx0, k, k)

    for g in range(NGROUP):
        idx_v = idx_bufs[g % 2]
        nidx = idx_bufs[(g + 1) % 2]
        if g + 1 < NGROUP:
            stage_idx(g + 1, nidx)

        @pl.loop(0, GROUP - NSLOT, step=NSLOT)
        def _seq(s, idx_v=idx_v):
            for k in range(NSLOT):
                i = s + k
                drain(k)
                issue(idx_v, i + NSLOT - 1, (k + NSLOT - 1) % NSLOT)
                reduce(k, i)

        if g + 1 < NGROUP:
            wait_idx(nidx)
        for k in range(NSLOT):
            i = GROUP - NSLOT + k
            drain(i % NSLOT)
            j = i + NSLOT - 1
            if j < GROUP:
                issue(idx_v, j, j % NSLOT)
            elif g + 1 < NGROUP:
                issue(nidx, j - GROUP, j % NSLOT)
            reduce(i % NSLOT, i)

        pltpu.sync_copy(out_v, sums_hbm.at[pl.ds(seq0 + g * GROUP, GROUP)])


@functools.cache
def _sc_sum():
    return pl.kernel(
        _sc_body,
        out_type=jax.ShapeDtypeStruct((B, DIM), jnp.float32),
        mesh=plsc.VectorSubcoreMesh(
            core_axis_name="c", subcore_axis_name="s",
            num_cores=NC, num_subcores=NS),
        compiler_params=pltpu.CompilerParams(
            needs_layout_passes=False, use_tc_tiling_on_sc=False),
        scratch_types=[
            pltpu.VMEM((GROUP * L,), jnp.int32),
            pltpu.VMEM((GROUP * L,), jnp.int32),
            pltpu.VMEM((L, PDIM), jnp.int32),
            pltpu.VMEM((L, PDIM), jnp.int32),
            pltpu.VMEM((L, PDIM), jnp.int32),
            pltpu.VMEM((L, PDIM), jnp.int32),
            pltpu.VMEM((GROUP, DIM), jnp.float32),
            pltpu.SemaphoreType.DMA,
            pltpu.SemaphoreType.DMA,
            pltpu.SemaphoreType.DMA,
            pltpu.SemaphoreType.DMA,
            pltpu.SemaphoreType.DMA,
        ],
    )


BLK = 2048


def _mlp_body(x_ref, sums_ref, w1_ref, b1_ref, w2_ref, b2_ref, out_ref):
    xb = x_ref[...]
    lengths = jnp.sum((xb != 0).astype(jnp.float32), axis=1, keepdims=True)
    pooled = sums_ref[...] / lengths
    h = jnp.dot(pooled, w1_ref[...], preferred_element_type=jnp.float32)
    h = jnp.maximum(h + b1_ref[...], 0.0)
    out_ref[...] = (jnp.dot(h, w2_ref[...], preferred_element_type=jnp.float32)
                    + b2_ref[...])


def _mlp(x2d, sums, w1, b1, w2, b2):
    return pl.pallas_call(
        _mlp_body,
        grid=(B // BLK,),
        in_specs=[
            pl.BlockSpec((BLK, L), lambda i: (i, 0)),
            pl.BlockSpec((BLK, DIM), lambda i: (i, 0)),
            pl.BlockSpec((DIM, HID), lambda i: (0, 0)),
            pl.BlockSpec((1, HID), lambda i: (0, 0)),
            pl.BlockSpec((HID, OUT), lambda i: (0, 0)),
            pl.BlockSpec((1, OUT), lambda i: (0, 0)),
        ],
        out_specs=pl.BlockSpec((BLK, OUT), lambda i: (i, 0)),
        out_shape=jax.ShapeDtypeStruct((B, OUT), jnp.float32),
    )(x2d, sums, w1, b1.reshape(1, HID), w2, b2.reshape(1, OUT))


def kernel(x, table, W1, b1, W2, b2):
    x32 = x.astype(jnp.int32)
    sums = _sc_sum()(x32.reshape(B * L), _pack_table(table))
    return _mlp(x32, sums, W1[_PERM, :], b1, W2, b2)


# speedup vs baseline: 2.0322x; 1.0077x over previous
import functools

import jax
import jax.numpy as jnp
import numpy as np
from jax import lax
from jax.experimental import pallas as pl
from jax.experimental.pallas import tpu as pltpu
from jax.experimental.pallas import tpu_sc as plsc

VOCAB = 100000
DIM = 128
B = 16384
L = 200
HID = 50
OUT = 4

NC = 2
NS = 16
NW = NC * NS
SEQ_PER_W = B // NW
GROUP = 32
NGROUP = SEQ_PER_W // GROUP
NSLOT = 8
PDIM = DIM // 2
PVR = PDIM // 16

_PERM = np.empty(DIM, np.int32)
for _c in range(PVR):
    for _k in range(16):
        _PERM[32 * _c + _k] = 16 * _c + _k
        _PERM[32 * _c + 16 + _k] = 64 + 16 * _c + _k


PACK_BLK = 2000


def _pack_body(t_ref, out_ref):
    u = jax.lax.bitcast_convert_type(t_ref[...], jnp.uint32)
    rb = (u + jnp.uint32(0x7FFF) + ((u >> 16) & jnp.uint32(1))) >> 16
    packed = (rb[:, PDIM:] << 16) | rb[:, :PDIM]
    out_ref[...] = jax.lax.bitcast_convert_type(packed, jnp.int32)


def _pack_table(table):
    return pl.pallas_call(
        _pack_body,
        grid=(VOCAB // PACK_BLK,),
        in_specs=[pl.BlockSpec((PACK_BLK, DIM), lambda i: (i, 0))],
        out_specs=pl.BlockSpec((PACK_BLK, PDIM), lambda i: (i, 0)),
        out_shape=jax.ShapeDtypeStruct((VOCAB, PDIM), jnp.int32),
    )(table)


def _sc_body(x_hbm, table_hbm, sums_hbm, idx,
             rows0, rows1, rows2, rows3, rows4, rows5, rows6, rows7, out_v,
             isem, sem0, sem1, sem2, sem3, sem4, sem5, sem6, sem7):
    wid = lax.axis_index("s") * NC + lax.axis_index("c")
    seq0 = wid * SEQ_PER_W
    rows = (rows0, rows1, rows2, rows3, rows4, rows5, rows6, rows7)
    sems = (sem0, sem1, sem2, sem3, sem4, sem5, sem6, sem7)

    def stage_idx(g, buf):
        pltpu.async_copy(
            x_hbm.at[pl.ds((seq0 + g * GROUP) * L, GROUP * L)], buf, isem)

    def wait_idx(buf):
        pltpu.make_async_copy(x_hbm.at[pl.ds(0, GROUP * L)], buf, isem).wait()

    def issue(idx_v, s_local, k):
        base = s_local * L
        pltpu.async_copy(table_hbm.at[idx_v.at[pl.ds(base, 128)]],
                         rows[k].at[pl.ds(0, 128)], sems[k])
        pltpu.async_copy(table_hbm.at[idx_v.at[pl.ds(base + 128, L - 128)]],
                         rows[k].at[pl.ds(128, L - 128)], sems[k])

    def drain(k):
        pltpu.make_async_copy(table_hbm.at[pl.ds(0, L)], rows[k], sems[k]).wait()

    def reduce(k, s_local):
        rows_ref = rows[k]

        def add_row(r, acc):
            out = list(acc)
            for c in range(PVR):
                v = rows_ref[r, pl.ds(16 * c, 16)]
                lo = plsc.bitcast(v << 16, jnp.float32)
                hi = plsc.bitcast(v & jnp.int32(-65536), jnp.float32)
                out[2 * c] = out[2 * c] + lo
                out[2 * c + 1] = out[2 * c + 1] + hi
            return out

        def body(r4, acc):
            for j in range(4):
                acc = add_row(4 * r4 + j, acc)
            return tuple(acc)

        acc = lax.fori_loop(
            0, L // 4, body,
            tuple(jnp.zeros((16,), jnp.float32) for _ in range(2 * PVR)))
        for c in range(PVR):
            out_v[s_local, pl.ds(32 * c, 16)] = acc[2 * c]
            out_v[s_local, pl.ds(32 * c + 16, 16)] = acc[2 * c + 1]

    stage_idx(0, idx.at[0])
    wait_idx(idx.at[0])
    for k in range(NSLOT - 1):
        issue(idx.at[0], k, k)

    @pl.loop(0, NGROUP)
    def _group(g):
        idx_v = idx.at[g % 2]
        nidx = idx.at[(g + 1) % 2]
        more = g + 1 < NGROUP

        @pl.when(more)
        def _():
            stage_idx(g + 1, nidx)

        @pl.loop(0, GROUP - NSLOT, step=NSLOT)
        def _seq(s):
            for k in range(NSLOT):
                i = s + k
                drain(k)
                issue(idx_v, i + NSLOT - 1, (k + NSLOT - 1) % NSLOT)
                reduce(k, i)

        @pl.when(more)
        def _():
            wait_idx(nidx)
        for k in range(NSLOT):
            i = GROUP - NSLOT + k
            drain(i % NSLOT)
            if k == 0:
                issue(idx_v, GROUP - 1, (GROUP - 1) % NSLOT)
            else:
                @pl.when(more)
                def _(k=k):
                    issue(nidx, k - 1, (k - 1) % NSLOT)
            reduce(i % NSLOT, i)

        pltpu.sync_copy(out_v, sums_hbm.at[pl.ds(seq0 + g * GROUP, GROUP)])


@functools.cache
def _sc_sum():
    return pl.kernel(
        _sc_body,
        out_type=jax.ShapeDtypeStruct((B, DIM), jnp.float32),
        mesh=plsc.VectorSubcoreMesh(
            core_axis_name="c", subcore_axis_name="s",
            num_cores=NC, num_subcores=NS),
        compiler_params=pltpu.CompilerParams(
            needs_layout_passes=False, use_tc_tiling_on_sc=False),
        scratch_types=[
            pltpu.VMEM((2, GROUP * L), jnp.int32),
            pltpu.VMEM((L, PDIM), jnp.int32),
            pltpu.VMEM((L, PDIM), jnp.int32),
            pltpu.VMEM((L, PDIM), jnp.int32),
            pltpu.VMEM((L, PDIM), jnp.int32),
            pltpu.VMEM((L, PDIM), jnp.int32),
            pltpu.VMEM((L, PDIM), jnp.int32),
            pltpu.VMEM((L, PDIM), jnp.int32),
            pltpu.VMEM((L, PDIM), jnp.int32),
            pltpu.VMEM((GROUP, DIM), jnp.float32),
        ] + [pltpu.SemaphoreType.DMA] * 9,
    )


BLK = 2048


def _mlp_body(x_ref, sums_ref, w1_ref, b1_ref, w2_ref, b2_ref, out_ref):
    xb = x_ref[...]
    lengths = jnp.sum((xb != 0).astype(jnp.float32), axis=1, keepdims=True)
    pooled = sums_ref[...] / lengths
    h = jnp.dot(pooled, w1_ref[...], preferred_element_type=jnp.float32)
    h = jnp.maximum(h + b1_ref[...], 0.0)
    out_ref[...] = (jnp.dot(h, w2_ref[...], preferred_element_type=jnp.float32)
                    + b2_ref[...])


def _mlp(x2d, sums, w1, b1, w2, b2):
    return pl.pallas_call(
        _mlp_body,
        grid=(B // BLK,),
        in_specs=[
            pl.BlockSpec((BLK, L), lambda i: (i, 0)),
            pl.BlockSpec((BLK, DIM), lambda i: (i, 0)),
            pl.BlockSpec((DIM, HID), lambda i: (0, 0)),
            pl.BlockSpec((1, HID), lambda i: (0, 0)),
            pl.BlockSpec((HID, OUT), lambda i: (0, 0)),
            pl.BlockSpec((1, OUT), lambda i: (0, 0)),
        ],
        out_specs=pl.BlockSpec((BLK, OUT), lambda i: (i, 0)),
        out_shape=jax.ShapeDtypeStruct((B, OUT), jnp.float32),
    )(x2d, sums, w1, b1.reshape(1, HID), w2, b2.reshape(1, OUT))


def kernel(x, table, W1, b1, W2, b2):
    x32 = x.astype(jnp.int32)
    sums = _sc_sum()(x32.reshape(B * L), _pack_table(table))
    return _mlp(x32, sums, W1[_PERM, :], b1, W2, b2)


# speedup vs baseline: 2.2354x; 1.1000x over previous
import functools

import jax
import jax.numpy as jnp
import numpy as np
from jax import lax
from jax.experimental import pallas as pl
from jax.experimental.pallas import tpu as pltpu
from jax.experimental.pallas import tpu_sc as plsc

VOCAB = 100000
DIM = 128
B = 16384
L = 200
HID = 50
OUT = 4

NC = 2
NS = 16
NW = NC * NS
SEQ_PER_W = B // NW
GROUP = 32
NGROUP = SEQ_PER_W // GROUP
NSLOT = 8
PDIM = DIM // 2
PVR = PDIM // 16

_PERM = np.empty(DIM, np.int32)
for _c in range(PVR):
    for _k in range(16):
        _PERM[32 * _c + _k] = 16 * _c + _k
        _PERM[32 * _c + 16 + _k] = 64 + 16 * _c + _k


PACK_BLK = 2000


def _pack_body(t_ref, out_ref):
    u = jax.lax.bitcast_convert_type(t_ref[...], jnp.uint32)
    rb = (u + jnp.uint32(0x7FFF) + ((u >> 16) & jnp.uint32(1))) >> 16
    packed = (rb[:, PDIM:] << 16) | rb[:, :PDIM]
    out_ref[...] = jax.lax.bitcast_convert_type(packed, jnp.int32)


def _pack_table(table):
    return pl.pallas_call(
        _pack_body,
        grid=(VOCAB // PACK_BLK,),
        in_specs=[pl.BlockSpec((PACK_BLK, DIM), lambda i: (i, 0))],
        out_specs=pl.BlockSpec((PACK_BLK, PDIM), lambda i: (i, 0)),
        out_shape=jax.ShapeDtypeStruct((VOCAB, PDIM), jnp.int32),
    )(table)


def _sc_body(x_hbm, table_hbm, sums_hbm, idx,
             rows0, rows1, rows2, rows3, rows4, rows5, rows6, rows7, out_v,
             isem, sem0, sem1, sem2, sem3, sem4, sem5, sem6, sem7):
    wid = lax.axis_index("s") * NC + lax.axis_index("c")
    seq0 = wid * SEQ_PER_W
    rows = (rows0, rows1, rows2, rows3, rows4, rows5, rows6, rows7)
    sems = (sem0, sem1, sem2, sem3, sem4, sem5, sem6, sem7)

    def stage_idx(g, buf):
        pltpu.async_copy(
            x_hbm.at[pl.ds((seq0 + g * GROUP) * L, GROUP * L)], buf, isem)

    def wait_idx(buf):
        pltpu.make_async_copy(x_hbm.at[pl.ds(0, GROUP * L)], buf, isem).wait()

    def issue(idx_v, s_local, k):
        base = s_local * L
        pltpu.async_copy(table_hbm.at[idx_v.at[pl.ds(base, 128)]],
                         rows[k].at[pl.ds(0, 128)], sems[k])
        pltpu.async_copy(table_hbm.at[idx_v.at[pl.ds(base + 128, L - 128)]],
                         rows[k].at[pl.ds(128, L - 128)], sems[k])

    def drain(k):
        pltpu.make_async_copy(table_hbm.at[pl.ds(0, L)], rows[k], sems[k]).wait()

    def reduce(k, s_local):
        rows_ref = rows[k]

        def add_pair(r, acc):
            out = list(acc)
            for c in range(PVR):
                va = rows_ref[r, pl.ds(16 * c, 16)]
                vb = rows_ref[r + 1, pl.ds(16 * c, 16)]
                s = (plsc.bitcast(va, jnp.bfloat16)
                     + plsc.bitcast(vb, jnp.bfloat16))
                d = plsc.bitcast(s, jnp.int32)
                lo = plsc.bitcast(d << 16, jnp.float32)
                hi = plsc.bitcast(d & jnp.int32(-65536), jnp.float32)
                out[2 * c] = out[2 * c] + lo
                out[2 * c + 1] = out[2 * c + 1] + hi
            return out

        def body(r4, acc):
            acc = add_pair(4 * r4, acc)
            acc = add_pair(4 * r4 + 2, acc)
            return tuple(acc)

        acc = lax.fori_loop(
            0, L // 4, body,
            tuple(jnp.zeros((16,), jnp.float32) for _ in range(2 * PVR)))
        for c in range(PVR):
            out_v[s_local, pl.ds(32 * c, 16)] = acc[2 * c]
            out_v[s_local, pl.ds(32 * c + 16, 16)] = acc[2 * c + 1]

    stage_idx(0, idx.at[0])
    wait_idx(idx.at[0])
    for k in range(NSLOT - 1):
        issue(idx.at[0], k, k)

    @pl.loop(0, NGROUP)
    def _group(g):
        idx_v = idx.at[g % 2]
        nidx = idx.at[(g + 1) % 2]
        more = g + 1 < NGROUP

        @pl.when(more)
        def _():
            stage_idx(g + 1, nidx)

        @pl.loop(0, GROUP - NSLOT, step=NSLOT)
        def _seq(s):
            for k in range(NSLOT):
                i = s + k
                drain(k)
                issue(idx_v, i + NSLOT - 1, (k + NSLOT - 1) % NSLOT)
                reduce(k, i)

        @pl.when(more)
        def _():
            wait_idx(nidx)
        for k in range(NSLOT):
            i = GROUP - NSLOT + k
            drain(i % NSLOT)
            if k == 0:
                issue(idx_v, GROUP - 1, (GROUP - 1) % NSLOT)
            else:
                @pl.when(more)
                def _(k=k):
                    issue(nidx, k - 1, (k - 1) % NSLOT)
            reduce(i % NSLOT, i)

        pltpu.sync_copy(out_v, sums_hbm.at[pl.ds(seq0 + g * GROUP, GROUP)])


@functools.cache
def _sc_sum():
    return pl.kernel(
        _sc_body,
        out_type=jax.ShapeDtypeStruct((B, DIM), jnp.float32),
        mesh=plsc.VectorSubcoreMesh(
            core_axis_name="c", subcore_axis_name="s",
            num_cores=NC, num_subcores=NS),
        compiler_params=pltpu.CompilerParams(
            needs_layout_passes=False, use_tc_tiling_on_sc=False),
        scratch_types=[
            pltpu.VMEM((2, GROUP * L), jnp.int32),
            pltpu.VMEM((L, PDIM), jnp.int32),
            pltpu.VMEM((L, PDIM), jnp.int32),
            pltpu.VMEM((L, PDIM), jnp.int32),
            pltpu.VMEM((L, PDIM), jnp.int32),
            pltpu.VMEM((L, PDIM), jnp.int32),
            pltpu.VMEM((L, PDIM), jnp.int32),
            pltpu.VMEM((L, PDIM), jnp.int32),
            pltpu.VMEM((L, PDIM), jnp.int32),
            pltpu.VMEM((GROUP, DIM), jnp.float32),
        ] + [pltpu.SemaphoreType.DMA] * 9,
    )


BLK = 2048


def _mlp_body(x_ref, sums_ref, w1_ref, b1_ref, w2_ref, b2_ref, out_ref):
    xb = x_ref[...]
    lengths = jnp.sum((xb != 0).astype(jnp.float32), axis=1, keepdims=True)
    pooled = sums_ref[...] / lengths
    h = jnp.dot(pooled, w1_ref[...], preferred_element_type=jnp.float32)
    h = jnp.maximum(h + b1_ref[...], 0.0)
    out_ref[...] = (jnp.dot(h, w2_ref[...], preferred_element_type=jnp.float32)
                    + b2_ref[...])


def _mlp(x2d, sums, w1, b1, w2, b2):
    return pl.pallas_call(
        _mlp_body,
        grid=(B // BLK,),
        in_specs=[
            pl.BlockSpec((BLK, L), lambda i: (i, 0)),
            pl.BlockSpec((BLK, DIM), lambda i: (i, 0)),
            pl.BlockSpec((DIM, HID), lambda i: (0, 0)),
            pl.BlockSpec((1, HID), lambda i: (0, 0)),
            pl.BlockSpec((HID, OUT), lambda i: (0, 0)),
            pl.BlockSpec((1, OUT), lambda i: (0, 0)),
        ],
        out_specs=pl.BlockSpec((BLK, OUT), lambda i: (i, 0)),
        out_shape=jax.ShapeDtypeStruct((B, OUT), jnp.float32),
    )(x2d, sums, w1, b1.reshape(1, HID), w2, b2.reshape(1, OUT))


def kernel(x, table, W1, b1, W2, b2):
    x32 = x.astype(jnp.int32)
    sums = _sc_sum()(x32.reshape(B * L), _pack_table(table))
    return _mlp(x32, sums, W1[_PERM, :], b1, W2, b2)


# speedup vs baseline: 2.3369x; 1.0454x over previous
import functools

import jax
import jax.numpy as jnp
import numpy as np
from jax import lax
from jax.experimental import pallas as pl
from jax.experimental.pallas import tpu as pltpu
from jax.experimental.pallas import tpu_sc as plsc

VOCAB = 100000
DIM = 128
B = 16384
L = 200
HID = 50
OUT = 4

NC = 2
NS = 16
NW = NC * NS
SEQ_PER_W = B // NW
GROUP = 32
NGROUP = SEQ_PER_W // GROUP
NSLOT = 8
PDIM = DIM // 2
PVR = PDIM // 16

_PERM = np.empty(DIM, np.int32)
for _c in range(PVR):
    for _k in range(16):
        _PERM[32 * _c + _k] = 16 * _c + _k
        _PERM[32 * _c + 16 + _k] = 64 + 16 * _c + _k


PACK_BLK = 10000


def _pack_body(t_ref, out_ref):
    u = jax.lax.bitcast_convert_type(t_ref[...], jnp.uint32)
    rb = (u + jnp.uint32(0x7FFF) + ((u >> 16) & jnp.uint32(1))) >> 16
    packed = (rb[:, PDIM:] << 16) | rb[:, :PDIM]
    out_ref[...] = jax.lax.bitcast_convert_type(packed, jnp.int32)


def _pack_table(table):
    return pl.pallas_call(
        _pack_body,
        grid=(VOCAB // PACK_BLK,),
        in_specs=[pl.BlockSpec((PACK_BLK, DIM), lambda i: (i, 0))],
        out_specs=pl.BlockSpec((PACK_BLK, PDIM), lambda i: (i, 0)),
        out_shape=jax.ShapeDtypeStruct((VOCAB, PDIM), jnp.int32),
    )(table)


def _sc_body(x_hbm, table_hbm, sums_hbm, idx,
             rows0, rows1, rows2, rows3, rows4, rows5, rows6, rows7, out_v,
             isem, sem0, sem1, sem2, sem3, sem4, sem5, sem6, sem7):
    wid = lax.axis_index("s") * NC + lax.axis_index("c")
    seq0 = wid * SEQ_PER_W
    rows = (rows0, rows1, rows2, rows3, rows4, rows5, rows6, rows7)
    sems = (sem0, sem1, sem2, sem3, sem4, sem5, sem6, sem7)

    def stage_idx(g, buf):
        pltpu.async_copy(
            x_hbm.at[pl.ds(seq0 + g * GROUP, GROUP)], buf, isem)

    def wait_idx(buf):
        pltpu.make_async_copy(x_hbm.at[pl.ds(0, GROUP)], buf, isem).wait()

    def issue(idx_v, s_local, k):
        pltpu.async_copy(table_hbm.at[idx_v.at[s_local, pl.ds(0, 128)]],
                         rows[k].at[pl.ds(0, 128)], sems[k])
        pltpu.async_copy(table_hbm.at[idx_v.at[s_local, pl.ds(128, L - 128)]],
                         rows[k].at[pl.ds(128, L - 128)], sems[k])

    def drain(k):
        pltpu.make_async_copy(table_hbm.at[pl.ds(0, L)], rows[k], sems[k]).wait()

    def reduce(k, s_local):
        rows_ref = rows[k]

        def add_pair(r, acc):
            out = list(acc)
            for c in range(PVR):
                va = rows_ref[r, pl.ds(16 * c, 16)]
                vb = rows_ref[r + 1, pl.ds(16 * c, 16)]
                s = (plsc.bitcast(va, jnp.bfloat16)
                     + plsc.bitcast(vb, jnp.bfloat16))
                d = plsc.bitcast(s, jnp.int32)
                lo = plsc.bitcast(d << 16, jnp.float32)
                hi = plsc.bitcast(d & jnp.int32(-65536), jnp.float32)
                out[2 * c] = out[2 * c] + lo
                out[2 * c + 1] = out[2 * c + 1] + hi
            return out

        def body(r4, acc):
            acc = add_pair(4 * r4, acc)
            acc = add_pair(4 * r4 + 2, acc)
            return tuple(acc)

        acc = lax.fori_loop(
            0, L // 4, body,
            tuple(jnp.zeros((16,), jnp.float32) for _ in range(2 * PVR)))
        for c in range(PVR):
            out_v[s_local, pl.ds(32 * c, 16)] = acc[2 * c]
            out_v[s_local, pl.ds(32 * c + 16, 16)] = acc[2 * c + 1]

    stage_idx(0, idx.at[0])
    wait_idx(idx.at[0])
    for k in range(NSLOT - 1):
        issue(idx.at[0], k, k)

    @pl.loop(0, NGROUP)
    def _group(g):
        idx_v = idx.at[g % 2]
        nidx = idx.at[(g + 1) % 2]
        more = g + 1 < NGROUP

        @pl.when(more)
        def _():
            stage_idx(g + 1, nidx)

        @pl.loop(0, GROUP - NSLOT, step=NSLOT)
        def _seq(s):
            for k in range(NSLOT):
                i = s + k
                drain(k)
                issue(idx_v, i + NSLOT - 1, (k + NSLOT - 1) % NSLOT)
                reduce(k, i)

        @pl.when(more)
        def _():
            wait_idx(nidx)
        for k in range(NSLOT):
            i = GROUP - NSLOT + k
            drain(i % NSLOT)
            if k == 0:
                issue(idx_v, GROUP - 1, (GROUP - 1) % NSLOT)
            else:
                @pl.when(more)
                def _(k=k):
                    issue(nidx, k - 1, (k - 1) % NSLOT)
            reduce(i % NSLOT, i)

        pltpu.sync_copy(out_v, sums_hbm.at[pl.ds(seq0 + g * GROUP, GROUP)])


@functools.cache
def _sc_sum():
    return pl.kernel(
        _sc_body,
        out_type=jax.ShapeDtypeStruct((B, DIM), jnp.float32),
        mesh=plsc.VectorSubcoreMesh(
            core_axis_name="c", subcore_axis_name="s",
            num_cores=NC, num_subcores=NS),
        compiler_params=pltpu.CompilerParams(
            needs_layout_passes=False, use_tc_tiling_on_sc=False),
        scratch_types=[
            pltpu.VMEM((2, GROUP, L), jnp.int32),
            pltpu.VMEM((L, PDIM), jnp.int32),
            pltpu.VMEM((L, PDIM), jnp.int32),
            pltpu.VMEM((L, PDIM), jnp.int32),
            pltpu.VMEM((L, PDIM), jnp.int32),
            pltpu.VMEM((L, PDIM), jnp.int32),
            pltpu.VMEM((L, PDIM), jnp.int32),
            pltpu.VMEM((L, PDIM), jnp.int32),
            pltpu.VMEM((L, PDIM), jnp.int32),
            pltpu.VMEM((GROUP, DIM), jnp.float32),
        ] + [pltpu.SemaphoreType.DMA] * 9,
    )


BLK = 2048


def _mlp_body(x_ref, sums_ref, w1_ref, b1_ref, w2_ref, b2_ref, out_ref):
    xb = x_ref[...]
    lengths = jnp.sum((xb != 0).astype(jnp.float32), axis=1, keepdims=True)
    pooled = sums_ref[...] / lengths
    h = jnp.dot(pooled, w1_ref[...], preferred_element_type=jnp.float32)
    h = jnp.maximum(h + b1_ref[...], 0.0)
    out_ref[...] = (jnp.dot(h, w2_ref[...], preferred_element_type=jnp.float32)
                    + b2_ref[...])


def _mlp(x2d, sums, w1, b1, w2, b2):
    return pl.pallas_call(
        _mlp_body,
        grid=(B // BLK,),
        in_specs=[
            pl.BlockSpec((BLK, L), lambda i: (i, 0)),
            pl.BlockSpec((BLK, DIM), lambda i: (i, 0)),
            pl.BlockSpec((DIM, HID), lambda i: (0, 0)),
            pl.BlockSpec((1, HID), lambda i: (0, 0)),
            pl.BlockSpec((HID, OUT), lambda i: (0, 0)),
            pl.BlockSpec((1, OUT), lambda i: (0, 0)),
        ],
        out_specs=pl.BlockSpec((BLK, OUT), lambda i: (i, 0)),
        out_shape=jax.ShapeDtypeStruct((B, OUT), jnp.float32),
    )(x2d, sums, w1, b1.reshape(1, HID), w2, b2.reshape(1, OUT))


def kernel(x, table, W1, b1, W2, b2):
    x32 = x.astype(jnp.int32)
    sums = _sc_sum()(x32, _pack_table(table))
    return _mlp(x32, sums, W1[_PERM, :], b1, W2, b2)


# speedup vs baseline: 3.0382x; 1.3001x over previous
import functools

import jax
import jax.numpy as jnp
import numpy as np
from jax import lax
from jax.experimental import pallas as pl
from jax.experimental.pallas import tpu as pltpu
from jax.experimental.pallas import tpu_sc as plsc

VOCAB = 100000
DIM = 128
B = 16384
L = 200
HID = 50
OUT = 4

NC = 2
NS = 16
NW = NC * NS
SEQ_PER_W = B // NW
GROUP = 32
NGROUP = SEQ_PER_W // GROUP
NSLOT = 8
HID2 = 64
PDIM = HID2 // 2
PVR = PDIM // 16

_PERM = np.empty(HID2, np.int32)
for _c in range(PVR):
    for _k in range(16):
        _PERM[32 * _c + _k] = 16 * _c + _k
        _PERM[32 * _c + 16 + _k] = PDIM + 16 * _c + _k


PACK_BLK = 10000


def _pack_body(t_ref, w1_ref, out_ref):
    p = lax.dot_general(
        t_ref[...], w1_ref[...], (((1,), (0,)), ((), ())),
        precision=lax.Precision.HIGHEST, preferred_element_type=jnp.float32)
    u = jax.lax.bitcast_convert_type(p, jnp.uint32)
    rb = (u + jnp.uint32(0x7FFF) + ((u >> 16) & jnp.uint32(1))) >> 16
    packed = (rb[:, PDIM:] << 16) | rb[:, :PDIM]
    out_ref[...] = jax.lax.bitcast_convert_type(packed, jnp.int32)


def _pack_table(table, w1p):
    return pl.pallas_call(
        _pack_body,
        grid=(VOCAB // PACK_BLK,),
        in_specs=[
            pl.BlockSpec((PACK_BLK, DIM), lambda i: (i, 0)),
            pl.BlockSpec((DIM, HID2), lambda i: (0, 0)),
        ],
        out_specs=pl.BlockSpec((PACK_BLK, PDIM), lambda i: (i, 0)),
        out_shape=jax.ShapeDtypeStruct((VOCAB, PDIM), jnp.int32),
    )(table, w1p)


def _sc_body(x_hbm, table_hbm, sums_hbm, idx,
             rows0, rows1, rows2, rows3, rows4, rows5, rows6, rows7, out_v,
             isem, sem0, sem1, sem2, sem3, sem4, sem5, sem6, sem7):
    wid = lax.axis_index("s") * NC + lax.axis_index("c")
    seq0 = wid * SEQ_PER_W
    rows = (rows0, rows1, rows2, rows3, rows4, rows5, rows6, rows7)
    sems = (sem0, sem1, sem2, sem3, sem4, sem5, sem6, sem7)

    def stage_idx(g, buf):
        pltpu.async_copy(
            x_hbm.at[pl.ds(seq0 + g * GROUP, GROUP)], buf, isem)

    def wait_idx(buf):
        pltpu.make_async_copy(x_hbm.at[pl.ds(0, GROUP)], buf, isem).wait()

    def issue(idx_v, s_local, k):
        pltpu.async_copy(table_hbm.at[idx_v.at[s_local, pl.ds(0, 128)]],
                         rows[k].at[pl.ds(0, 128)], sems[k])
        pltpu.async_copy(table_hbm.at[idx_v.at[s_local, pl.ds(128, L - 128)]],
                         rows[k].at[pl.ds(128, L - 128)], sems[k])

    def drain(k):
        pltpu.make_async_copy(table_hbm.at[pl.ds(0, L)], rows[k], sems[k]).wait()

    def reduce(k, s_local):
        rows_ref = rows[k]

        def add_pair(r, acc):
            out = list(acc)
            for c in range(PVR):
                va = rows_ref[r, pl.ds(16 * c, 16)]
                vb = rows_ref[r + 1, pl.ds(16 * c, 16)]
                s = (plsc.bitcast(va, jnp.bfloat16)
                     + plsc.bitcast(vb, jnp.bfloat16))
                d = plsc.bitcast(s, jnp.int32)
                lo = plsc.bitcast(d << 16, jnp.float32)
                hi = plsc.bitcast(d & jnp.int32(-65536), jnp.float32)
                out[2 * c] = out[2 * c] + lo
                out[2 * c + 1] = out[2 * c + 1] + hi
            return out

        def body(r4, acc):
            acc = add_pair(4 * r4, acc)
            acc = add_pair(4 * r4 + 2, acc)
            return tuple(acc)

        acc = lax.fori_loop(
            0, L // 4, body,
            tuple(jnp.zeros((16,), jnp.float32) for _ in range(2 * PVR)))
        for c in range(PVR):
            out_v[s_local, pl.ds(32 * c, 16)] = acc[2 * c]
            out_v[s_local, pl.ds(32 * c + 16, 16)] = acc[2 * c + 1]

    stage_idx(0, idx.at[0])
    wait_idx(idx.at[0])
    for k in range(NSLOT - 1):
        issue(idx.at[0], k, k)

    @pl.loop(0, NGROUP)
    def _group(g):
        idx_v = idx.at[g % 2]
        nidx = idx.at[(g + 1) % 2]
        more = g + 1 < NGROUP

        @pl.when(more)
        def _():
            stage_idx(g + 1, nidx)

        @pl.loop(0, GROUP - NSLOT, step=NSLOT)
        def _seq(s):
            for k in range(NSLOT):
                i = s + k
                drain(k)
                issue(idx_v, i + NSLOT - 1, (k + NSLOT - 1) % NSLOT)
                reduce(k, i)

        @pl.when(more)
        def _():
            wait_idx(nidx)
        for k in range(NSLOT):
            i = GROUP - NSLOT + k
            drain(i % NSLOT)
            if k == 0:
                issue(idx_v, GROUP - 1, (GROUP - 1) % NSLOT)
            else:
                @pl.when(more)
                def _(k=k):
                    issue(nidx, k - 1, (k - 1) % NSLOT)
            reduce(i % NSLOT, i)

        pltpu.sync_copy(out_v, sums_hbm.at[pl.ds(seq0 + g * GROUP, GROUP)])


@functools.cache
def _sc_sum():
    return pl.kernel(
        _sc_body,
        out_type=jax.ShapeDtypeStruct((B, HID2), jnp.float32),
        mesh=plsc.VectorSubcoreMesh(
            core_axis_name="c", subcore_axis_name="s",
            num_cores=NC, num_subcores=NS),
        compiler_params=pltpu.CompilerParams(
            needs_layout_passes=False, use_tc_tiling_on_sc=False),
        scratch_types=[
            pltpu.VMEM((2, GROUP, L), jnp.int32),
            pltpu.VMEM((L, PDIM), jnp.int32),
            pltpu.VMEM((L, PDIM), jnp.int32),
            pltpu.VMEM((L, PDIM), jnp.int32),
            pltpu.VMEM((L, PDIM), jnp.int32),
            pltpu.VMEM((L, PDIM), jnp.int32),
            pltpu.VMEM((L, PDIM), jnp.int32),
            pltpu.VMEM((L, PDIM), jnp.int32),
            pltpu.VMEM((L, PDIM), jnp.int32),
            pltpu.VMEM((GROUP, HID2), jnp.float32),
        ] + [pltpu.SemaphoreType.DMA] * 9,
    )


BLK = 2048


def _mlp_body(x_ref, sums_ref, b1_ref, w2_ref, b2_ref, out_ref):
    xb = x_ref[...]
    lengths = jnp.sum((xb != 0).astype(jnp.float32), axis=1, keepdims=True)
    h = jnp.maximum(sums_ref[...] / lengths + b1_ref[...], 0.0)
    out_ref[...] = (jnp.dot(h, w2_ref[...], preferred_element_type=jnp.float32)
                    + b2_ref[...])


def _mlp(x2d, sums, b1p, w2p, b2):
    return pl.pallas_call(
        _mlp_body,
        grid=(B // BLK,),
        in_specs=[
            pl.BlockSpec((BLK, L), lambda i: (i, 0)),
            pl.BlockSpec((BLK, HID2), lambda i: (i, 0)),
            pl.BlockSpec((1, HID2), lambda i: (0, 0)),
            pl.BlockSpec((HID2, OUT), lambda i: (0, 0)),
            pl.BlockSpec((1, OUT), lambda i: (0, 0)),
        ],
        out_specs=pl.BlockSpec((BLK, OUT), lambda i: (i, 0)),
        out_shape=jax.ShapeDtypeStruct((B, OUT), jnp.float32),
    )(x2d, sums, b1p.reshape(1, HID2), w2p, b2.reshape(1, OUT))


def kernel(x, table, W1, b1, W2, b2):
    x32 = x.astype(jnp.int32)
    w1p = jnp.zeros((DIM, HID2), jnp.float32).at[:, :HID].set(W1)
    b1p = jnp.zeros((HID2,), jnp.float32).at[:HID].set(b1)[_PERM]
    w2p = jnp.zeros((HID2, OUT), jnp.float32).at[:HID, :].set(W2)[_PERM, :]
    sums = _sc_sum()(x32, _pack_table(table, w1p))
    return _mlp(x32, sums, b1p, w2p, b2)


# speedup vs baseline: 3.0524x; 1.0047x over previous
import functools

import jax
import jax.numpy as jnp
import numpy as np
from jax import lax
from jax.experimental import pallas as pl
from jax.experimental.pallas import tpu as pltpu
from jax.experimental.pallas import tpu_sc as plsc

VOCAB = 100000
DIM = 128
B = 16384
L = 200
HID = 50
OUT = 4

NC = 2
NS = 16
NW = NC * NS
SEQ_PER_W = B // NW
GROUP = 32
NGROUP = SEQ_PER_W // GROUP
NSLOT = 8
HID2 = 64
PDIM = HID2 // 2
PVR = PDIM // 16

_PERM = np.empty(HID2, np.int32)
for _c in range(PVR):
    for _k in range(16):
        _PERM[32 * _c + _k] = 16 * _c + _k
        _PERM[32 * _c + 16 + _k] = PDIM + 16 * _c + _k


PACK_BLK = 2000


def _pack_body(t_ref, w1_ref, out_ref):
    p = lax.dot_general(
        t_ref[...], w1_ref[...], (((1,), (0,)), ((), ())),
        preferred_element_type=jnp.float32)
    u = jax.lax.bitcast_convert_type(p, jnp.uint32)
    rb = (u + jnp.uint32(0x7FFF) + ((u >> 16) & jnp.uint32(1))) >> 16
    packed = (rb[:, PDIM:] << 16) | rb[:, :PDIM]
    out_ref[...] = jax.lax.bitcast_convert_type(packed, jnp.int32)


def _pack_table(table, w1p):
    return pl.pallas_call(
        _pack_body,
        grid=(VOCAB // PACK_BLK,),
        in_specs=[
            pl.BlockSpec((PACK_BLK, DIM), lambda i: (i, 0)),
            pl.BlockSpec((DIM, HID2), lambda i: (0, 0)),
        ],
        out_specs=pl.BlockSpec((PACK_BLK, PDIM), lambda i: (i, 0)),
        out_shape=jax.ShapeDtypeStruct((VOCAB, PDIM), jnp.int32),
    )(table, w1p)


def _sc_body(x_hbm, table_hbm, sums_hbm, idx,
             rows0, rows1, rows2, rows3, rows4, rows5, rows6, rows7, out_v,
             isem, sem0, sem1, sem2, sem3, sem4, sem5, sem6, sem7):
    wid = lax.axis_index("s") * NC + lax.axis_index("c")
    seq0 = wid * SEQ_PER_W
    rows = (rows0, rows1, rows2, rows3, rows4, rows5, rows6, rows7)
    sems = (sem0, sem1, sem2, sem3, sem4, sem5, sem6, sem7)

    def stage_idx(g, buf):
        pltpu.async_copy(
            x_hbm.at[pl.ds(seq0 + g * GROUP, GROUP)], buf, isem)

    def wait_idx(buf):
        pltpu.make_async_copy(x_hbm.at[pl.ds(0, GROUP)], buf, isem).wait()

    def issue(idx_v, s_local, k):
        pltpu.async_copy(table_hbm.at[idx_v.at[s_local, pl.ds(0, 128)]],
                         rows[k].at[pl.ds(0, 128)], sems[k])
        pltpu.async_copy(table_hbm.at[idx_v.at[s_local, pl.ds(128, L - 128)]],
                         rows[k].at[pl.ds(128, L - 128)], sems[k])

    def drain(k):
        pltpu.make_async_copy(table_hbm.at[pl.ds(0, L)], rows[k], sems[k]).wait()

    def reduce(k, s_local):
        rows_ref = rows[k]

        def add_pair(r, acc):
            out = list(acc)
            for c in range(PVR):
                va = rows_ref[r, pl.ds(16 * c, 16)]
                vb = rows_ref[r + 1, pl.ds(16 * c, 16)]
                s = (plsc.bitcast(va, jnp.bfloat16)
                     + plsc.bitcast(vb, jnp.bfloat16))
                d = plsc.bitcast(s, jnp.int32)
                lo = plsc.bitcast(d << 16, jnp.float32)
                hi = plsc.bitcast(d & jnp.int32(-65536), jnp.float32)
                out[2 * c] = out[2 * c] + lo
                out[2 * c + 1] = out[2 * c + 1] + hi
            return out

        def body(r4, acc):
            acc = add_pair(4 * r4, acc)
            acc = add_pair(4 * r4 + 2, acc)
            return tuple(acc)

        acc = lax.fori_loop(
            0, L // 4, body,
            tuple(jnp.zeros((16,), jnp.float32) for _ in range(2 * PVR)))
        for c in range(PVR):
            out_v[s_local, pl.ds(32 * c, 16)] = acc[2 * c]
            out_v[s_local, pl.ds(32 * c + 16, 16)] = acc[2 * c + 1]

    stage_idx(0, idx.at[0])
    wait_idx(idx.at[0])
    for k in range(NSLOT - 1):
        issue(idx.at[0], k, k)

    @pl.loop(0, NGROUP)
    def _group(g):
        idx_v = idx.at[g % 2]
        nidx = idx.at[(g + 1) % 2]
        more = g + 1 < NGROUP

        @pl.when(more)
        def _():
            stage_idx(g + 1, nidx)

        @pl.loop(0, GROUP - NSLOT, step=NSLOT)
        def _seq(s):
            for k in range(NSLOT):
                i = s + k
                drain(k)
                issue(idx_v, i + NSLOT - 1, (k + NSLOT - 1) % NSLOT)
                reduce(k, i)

        @pl.when(more)
        def _():
            wait_idx(nidx)
        for k in range(NSLOT):
            i = GROUP - NSLOT + k
            drain(i % NSLOT)
            if k == 0:
                issue(idx_v, GROUP - 1, (GROUP - 1) % NSLOT)
            else:
                @pl.when(more)
                def _(k=k):
                    issue(nidx, k - 1, (k - 1) % NSLOT)
            reduce(i % NSLOT, i)

        pltpu.sync_copy(out_v, sums_hbm.at[pl.ds(seq0 + g * GROUP, GROUP)])


@functools.cache
def _sc_sum():
    return pl.kernel(
        _sc_body,
        out_type=jax.ShapeDtypeStruct((B, HID2), jnp.float32),
        mesh=plsc.VectorSubcoreMesh(
            core_axis_name="c", subcore_axis_name="s",
            num_cores=NC, num_subcores=NS),
        compiler_params=pltpu.CompilerParams(
            needs_layout_passes=False, use_tc_tiling_on_sc=False),
        scratch_types=[
            pltpu.VMEM((2, GROUP, L), jnp.int32),
            pltpu.VMEM((L, PDIM), jnp.int32),
            pltpu.VMEM((L, PDIM), jnp.int32),
            pltpu.VMEM((L, PDIM), jnp.int32),
            pltpu.VMEM((L, PDIM), jnp.int32),
            pltpu.VMEM((L, PDIM), jnp.int32),
            pltpu.VMEM((L, PDIM), jnp.int32),
            pltpu.VMEM((L, PDIM), jnp.int32),
            pltpu.VMEM((L, PDIM), jnp.int32),
            pltpu.VMEM((GROUP, HID2), jnp.float32),
        ] + [pltpu.SemaphoreType.DMA] * 9,
    )


BLK = 2048


def _mlp_body(x_ref, sums_ref, b1_ref, w2_ref, b2_ref, out_ref):
    xb = x_ref[...]
    lengths = jnp.sum((xb != 0).astype(jnp.float32), axis=1, keepdims=True)
    h = jnp.maximum(sums_ref[...] / lengths + b1_ref[...], 0.0)
    out_ref[...] = (jnp.dot(h, w2_ref[...], preferred_element_type=jnp.float32)
                    + b2_ref[...])


def _mlp(x2d, sums, b1p, w2p, b2):
    return pl.pallas_call(
        _mlp_body,
        grid=(B // BLK,),
        in_specs=[
            pl.BlockSpec((BLK, L), lambda i: (i, 0)),
            pl.BlockSpec((BLK, HID2), lambda i: (i, 0)),
            pl.BlockSpec((1, HID2), lambda i: (0, 0)),
            pl.BlockSpec((HID2, OUT), lambda i: (0, 0)),
            pl.BlockSpec((1, OUT), lambda i: (0, 0)),
        ],
        out_specs=pl.BlockSpec((BLK, OUT), lambda i: (i, 0)),
        out_shape=jax.ShapeDtypeStruct((B, OUT), jnp.float32),
    )(x2d, sums, b1p.reshape(1, HID2), w2p, b2.reshape(1, OUT))


def kernel(x, table, W1, b1, W2, b2):
    x32 = x.astype(jnp.int32)
    w1p = jnp.zeros((DIM, HID2), jnp.float32).at[:, :HID].set(W1)
    b1p = jnp.zeros((HID2,), jnp.float32).at[:HID].set(b1)[_PERM]
    w2p = jnp.zeros((HID2, OUT), jnp.float32).at[:HID, :].set(W2)[_PERM, :]
    sums = _sc_sum()(x32, _pack_table(table, w1p))
    return _mlp(x32, sums, b1p, w2p, b2)


# speedup vs baseline: 3.0846x; 1.0106x over previous
import functools

import jax
import jax.numpy as jnp
import numpy as np
from jax import lax
from jax.experimental import pallas as pl
from jax.experimental.pallas import tpu as pltpu
from jax.experimental.pallas import tpu_sc as plsc

VOCAB = 100000
DIM = 128
B = 16384
L = 200
HID = 50
OUT = 4

NC = 2
NS = 16
NW = NC * NS
SEQ_PER_W = B // NW
GROUP = 32
NGROUP = SEQ_PER_W // GROUP
NSLOT = 8
HID2 = 64
PDIM = HID2 // 2
PVR = PDIM // 16

_PERM = np.empty(HID2, np.int32)
for _c in range(PVR):
    for _k in range(16):
        _PERM[32 * _c + _k] = 16 * _c + _k
        _PERM[32 * _c + 16 + _k] = PDIM + 16 * _c + _k


PACK_BLK = 2000


def _pack_body(t_ref, w1_ref, out_ref):
    p = lax.dot_general(
        t_ref[...], w1_ref[...], (((1,), (0,)), ((), ())),
        preferred_element_type=jnp.float32)
    u = jax.lax.bitcast_convert_type(p, jnp.uint32)
    rb = (u + jnp.uint32(0x7FFF) + ((u >> 16) & jnp.uint32(1))) >> 16
    packed = (rb[:, PDIM:] << 16) | rb[:, :PDIM]
    out_ref[...] = jax.lax.bitcast_convert_type(packed, jnp.int32)


def _pack_table(table, w1p):
    return pl.pallas_call(
        _pack_body,
        grid=(VOCAB // PACK_BLK,),
        in_specs=[
            pl.BlockSpec((PACK_BLK, DIM), lambda i: (i, 0)),
            pl.BlockSpec((DIM, HID2), lambda i: (0, 0)),
        ],
        out_specs=pl.BlockSpec((PACK_BLK, PDIM), lambda i: (i, 0)),
        out_shape=jax.ShapeDtypeStruct((VOCAB, PDIM), jnp.int32),
    )(table, w1p)


def _sc_body(x_hbm, table_hbm, sums_hbm, idx,
             rows0, rows1, rows2, rows3, rows4, rows5, rows6, rows7, out_v,
             isem, sem0, sem1, sem2, sem3, sem4, sem5, sem6, sem7, osem):
    wid = lax.axis_index("s") * NC + lax.axis_index("c")
    seq0 = wid * SEQ_PER_W
    rows = (rows0, rows1, rows2, rows3, rows4, rows5, rows6, rows7)
    sems = (sem0, sem1, sem2, sem3, sem4, sem5, sem6, sem7)

    def stage_idx(g, buf):
        pltpu.async_copy(
            x_hbm.at[pl.ds(seq0 + g * GROUP, GROUP)], buf, isem)

    def wait_idx(buf):
        pltpu.make_async_copy(x_hbm.at[pl.ds(0, GROUP)], buf, isem).wait()

    def issue(idx_v, s_local, k):
        pltpu.async_copy(table_hbm.at[idx_v.at[s_local, pl.ds(0, 128)]],
                         rows[k].at[pl.ds(0, 128)], sems[k])
        pltpu.async_copy(table_hbm.at[idx_v.at[s_local, pl.ds(128, L - 128)]],
                         rows[k].at[pl.ds(128, L - 128)], sems[k])

    def drain(k):
        pltpu.make_async_copy(table_hbm.at[pl.ds(0, L)], rows[k], sems[k]).wait()

    def reduce(k, s_local, ob):
        rows_ref = rows[k]

        def add_pair(r, acc):
            out = list(acc)
            for c in range(PVR):
                va = rows_ref[r, pl.ds(16 * c, 16)]
                vb = rows_ref[r + 1, pl.ds(16 * c, 16)]
                s = (plsc.bitcast(va, jnp.bfloat16)
                     + plsc.bitcast(vb, jnp.bfloat16))
                d = plsc.bitcast(s, jnp.int32)
                lo = plsc.bitcast(d << 16, jnp.float32)
                hi = plsc.bitcast(d & jnp.int32(-65536), jnp.float32)
                out[2 * c] = out[2 * c] + lo
                out[2 * c + 1] = out[2 * c + 1] + hi
            return out

        def body(r8, acc):
            for j in range(4):
                acc = add_pair(8 * r8 + 2 * j, acc)
            return tuple(acc)

        acc = lax.fori_loop(
            0, L // 8, body,
            tuple(jnp.zeros((16,), jnp.float32) for _ in range(2 * PVR)))
        for c in range(PVR):
            ob[s_local, pl.ds(32 * c, 16)] = acc[2 * c]
            ob[s_local, pl.ds(32 * c + 16, 16)] = acc[2 * c + 1]

    stage_idx(0, idx.at[0])
    wait_idx(idx.at[0])
    for k in range(NSLOT - 1):
        issue(idx.at[0], k, k)

    @pl.loop(0, NGROUP)
    def _group(g):
        idx_v = idx.at[g % 2]
        nidx = idx.at[(g + 1) % 2]
        ob = out_v.at[g]
        more = g + 1 < NGROUP

        @pl.when(more)
        def _():
            stage_idx(g + 1, nidx)

        @pl.loop(0, GROUP - NSLOT, step=NSLOT)
        def _seq(s):
            for k in range(NSLOT):
                i = s + k
                drain(k)
                issue(idx_v, i + NSLOT - 1, (k + NSLOT - 1) % NSLOT)
                reduce(k, i, ob)

        @pl.when(more)
        def _():
            wait_idx(nidx)
        for k in range(NSLOT):
            i = GROUP - NSLOT + k
            drain(i % NSLOT)
            if k == 0:
                issue(idx_v, GROUP - 1, (GROUP - 1) % NSLOT)
            else:
                @pl.when(more)
                def _(k=k):
                    issue(nidx, k - 1, (k - 1) % NSLOT)
            reduce(i % NSLOT, i, ob)

        pltpu.async_copy(ob, sums_hbm.at[pl.ds(seq0 + g * GROUP, GROUP)], osem)

    @pl.loop(0, NGROUP)
    def _drain_out(g):
        pltpu.make_async_copy(
            out_v.at[0], sums_hbm.at[pl.ds(0, GROUP)], osem).wait()


@functools.cache
def _sc_sum():
    return pl.kernel(
        _sc_body,
        out_type=jax.ShapeDtypeStruct((B, HID2), jnp.float32),
        mesh=plsc.VectorSubcoreMesh(
            core_axis_name="c", subcore_axis_name="s",
            num_cores=NC, num_subcores=NS),
        compiler_params=pltpu.CompilerParams(
            needs_layout_passes=False, use_tc_tiling_on_sc=False),
        scratch_types=[
            pltpu.VMEM((2, GROUP, L), jnp.int32),
            pltpu.VMEM((L, PDIM), jnp.int32),
            pltpu.VMEM((L, PDIM), jnp.int32),
            pltpu.VMEM((L, PDIM), jnp.int32),
            pltpu.VMEM((L, PDIM), jnp.int32),
            pltpu.VMEM((L, PDIM), jnp.int32),
            pltpu.VMEM((L, PDIM), jnp.int32),
            pltpu.VMEM((L, PDIM), jnp.int32),
            pltpu.VMEM((L, PDIM), jnp.int32),
            pltpu.VMEM((NGROUP, GROUP, HID2), jnp.float32),
        ] + [pltpu.SemaphoreType.DMA] * 10,
    )


BLK = 2048


def _mlp_body(x_ref, sums_ref, b1_ref, w2_ref, b2_ref, out_ref):
    xb = x_ref[...]
    lengths = jnp.sum((xb != 0).astype(jnp.float32), axis=1, keepdims=True)
    h = jnp.maximum(sums_ref[...] / lengths + b1_ref[...], 0.0)
    out_ref[...] = (jnp.dot(h, w2_ref[...], preferred_element_type=jnp.float32)
                    + b2_ref[...])


def _mlp(x2d, sums, b1p, w2p, b2):
    return pl.pallas_call(
        _mlp_body,
        grid=(B // BLK,),
        in_specs=[
            pl.BlockSpec((BLK, L), lambda i: (i, 0)),
            pl.BlockSpec((BLK, HID2), lambda i: (i, 0)),
            pl.BlockSpec((1, HID2), lambda i: (0, 0)),
            pl.BlockSpec((HID2, OUT), lambda i: (0, 0)),
            pl.BlockSpec((1, OUT), lambda i: (0, 0)),
        ],
        out_specs=pl.BlockSpec((BLK, OUT), lambda i: (i, 0)),
        out_shape=jax.ShapeDtypeStruct((B, OUT), jnp.float32),
    )(x2d, sums, b1p.reshape(1, HID2), w2p, b2.reshape(1, OUT))


def kernel(x, table, W1, b1, W2, b2):
    x32 = x.astype(jnp.int32)
    w1p = jnp.zeros((DIM, HID2), jnp.float32).at[:, :HID].set(W1)
    b1p = jnp.zeros((HID2,), jnp.float32).at[:HID].set(b1)[_PERM]
    w2p = jnp.zeros((HID2, OUT), jnp.float32).at[:HID, :].set(W2)[_PERM, :]
    sums = _sc_sum()(x32, _pack_table(table, w1p))
    return _mlp(x32, sums, b1p, w2p, b2)


# speedup vs baseline: 3.0935x; 1.0029x over previous
import functools

import jax
import jax.numpy as jnp
import numpy as np
from jax import lax
from jax.experimental import pallas as pl
from jax.experimental.pallas import tpu as pltpu
from jax.experimental.pallas import tpu_sc as plsc

VOCAB = 100000
DIM = 128
B = 16384
L = 200
HID = 50
OUT = 4

NC = 2
NS = 16
NW = NC * NS
SEQ_PER_W = B // NW
GROUP = 32
NGROUP = SEQ_PER_W // GROUP
NSLOT = 8
HID2 = 64
PDIM = HID2 // 2
PVR = PDIM // 16

_PERM = np.empty(HID2, np.int32)
for _c in range(PVR):
    for _k in range(16):
        _PERM[32 * _c + _k] = 16 * _c + _k
        _PERM[32 * _c + 16 + _k] = PDIM + 16 * _c + _k


PACK_BLK = 2000


def _pack_body(t_ref, w1_ref, out_ref):
    p = lax.dot_general(
        t_ref[...], w1_ref[...], (((1,), (0,)), ((), ())),
        preferred_element_type=jnp.float32)
    u = jax.lax.bitcast_convert_type(p, jnp.uint32)
    rb = (u + jnp.uint32(0x7FFF) + ((u >> 16) & jnp.uint32(1))) >> 16
    packed = (rb[:, PDIM:] << 16) | rb[:, :PDIM]
    out_ref[...] = jax.lax.bitcast_convert_type(packed, jnp.int32)


def _pack_table(table, w1p):
    return pl.pallas_call(
        _pack_body,
        grid=(VOCAB // PACK_BLK,),
        in_specs=[
            pl.BlockSpec((PACK_BLK, DIM), lambda i: (i, 0)),
            pl.BlockSpec((DIM, HID2), lambda i: (0, 0)),
        ],
        out_specs=pl.BlockSpec((PACK_BLK, PDIM), lambda i: (i, 0)),
        out_shape=jax.ShapeDtypeStruct((VOCAB, PDIM), jnp.int32),
        compiler_params=pltpu.CompilerParams(
            dimension_semantics=("parallel",)),
    )(table, w1p)


def _sc_body(x_hbm, table_hbm, sums_hbm, idx,
             rows0, rows1, rows2, rows3, rows4, rows5, rows6, rows7, out_v,
             isem, sem0, sem1, sem2, sem3, sem4, sem5, sem6, sem7, osem):
    wid = lax.axis_index("s") * NC + lax.axis_index("c")
    seq0 = wid * SEQ_PER_W
    rows = (rows0, rows1, rows2, rows3, rows4, rows5, rows6, rows7)
    sems = (sem0, sem1, sem2, sem3, sem4, sem5, sem6, sem7)

    def stage_idx(g, buf):
        pltpu.async_copy(
            x_hbm.at[pl.ds(seq0 + g * GROUP, GROUP)], buf, isem)

    def wait_idx(buf):
        pltpu.make_async_copy(x_hbm.at[pl.ds(0, GROUP)], buf, isem).wait()

    def issue(idx_v, s_local, k):
        pltpu.async_copy(table_hbm.at[idx_v.at[s_local, pl.ds(0, 128)]],
                         rows[k].at[pl.ds(0, 128)], sems[k])
        pltpu.async_copy(table_hbm.at[idx_v.at[s_local, pl.ds(128, L - 128)]],
                         rows[k].at[pl.ds(128, L - 128)], sems[k])

    def drain(k):
        pltpu.make_async_copy(table_hbm.at[pl.ds(0, L)], rows[k], sems[k]).wait()

    def reduce(k, s_local, ob):
        rows_ref = rows[k]

        def add_pair(r, acc):
            out = list(acc)
            for c in range(PVR):
                va = rows_ref[r, pl.ds(16 * c, 16)]
                vb = rows_ref[r + 1, pl.ds(16 * c, 16)]
                s = (plsc.bitcast(va, jnp.bfloat16)
                     + plsc.bitcast(vb, jnp.bfloat16))
                d = plsc.bitcast(s, jnp.int32)
                lo = plsc.bitcast(d << 16, jnp.float32)
                hi = plsc.bitcast(d & jnp.int32(-65536), jnp.float32)
                out[2 * c] = out[2 * c] + lo
                out[2 * c + 1] = out[2 * c + 1] + hi
            return out

        def body(r8, acc):
            for j in range(4):
                acc = add_pair(8 * r8 + 2 * j, acc)
            return tuple(acc)

        acc = lax.fori_loop(
            0, L // 8, body,
            tuple(jnp.zeros((16,), jnp.float32) for _ in range(2 * PVR)))
        for c in range(PVR):
            ob[s_local, pl.ds(32 * c, 16)] = acc[2 * c]
            ob[s_local, pl.ds(32 * c + 16, 16)] = acc[2 * c + 1]

    stage_idx(0, idx.at[0])
    wait_idx(idx.at[0])
    for k in range(NSLOT - 1):
        issue(idx.at[0], k, k)

    @pl.loop(0, NGROUP)
    def _group(g):
        idx_v = idx.at[g % 2]
        nidx = idx.at[(g + 1) % 2]
        ob = out_v.at[g]
        more = g + 1 < NGROUP

        @pl.when(more)
        def _():
            stage_idx(g + 1, nidx)

        @pl.loop(0, GROUP - NSLOT, step=NSLOT)
        def _seq(s):
            for k in range(NSLOT):
                i = s + k
                drain(k)
                issue(idx_v, i + NSLOT - 1, (k + NSLOT - 1) % NSLOT)
                reduce(k, i, ob)

        @pl.when(more)
        def _():
            wait_idx(nidx)
        for k in range(NSLOT):
            i = GROUP - NSLOT + k
            drain(i % NSLOT)
            if k == 0:
                issue(idx_v, GROUP - 1, (GROUP - 1) % NSLOT)
            else:
                @pl.when(more)
                def _(k=k):
                    issue(nidx, k - 1, (k - 1) % NSLOT)
            reduce(i % NSLOT, i, ob)

        pltpu.async_copy(ob, sums_hbm.at[pl.ds(seq0 + g * GROUP, GROUP)], osem)

    @pl.loop(0, NGROUP)
    def _drain_out(g):
        pltpu.make_async_copy(
            out_v.at[0], sums_hbm.at[pl.ds(0, GROUP)], osem).wait()


@functools.cache
def _sc_sum():
    return pl.kernel(
        _sc_body,
        out_type=jax.ShapeDtypeStruct((B, HID2), jnp.float32),
        mesh=plsc.VectorSubcoreMesh(
            core_axis_name="c", subcore_axis_name="s",
            num_cores=NC, num_subcores=NS),
        compiler_params=pltpu.CompilerParams(
            needs_layout_passes=False, use_tc_tiling_on_sc=False),
        scratch_types=[
            pltpu.VMEM((2, GROUP, L), jnp.int32),
            pltpu.VMEM((L, PDIM), jnp.int32),
            pltpu.VMEM((L, PDIM), jnp.int32),
            pltpu.VMEM((L, PDIM), jnp.int32),
            pltpu.VMEM((L, PDIM), jnp.int32),
            pltpu.VMEM((L, PDIM), jnp.int32),
            pltpu.VMEM((L, PDIM), jnp.int32),
            pltpu.VMEM((L, PDIM), jnp.int32),
            pltpu.VMEM((L, PDIM), jnp.int32),
            pltpu.VMEM((NGROUP, GROUP, HID2), jnp.float32),
        ] + [pltpu.SemaphoreType.DMA] * 10,
    )


BLK = 2048


def _mlp_body(x_ref, sums_ref, b1_ref, w2_ref, b2_ref, out_ref):
    xb = x_ref[...]
    lengths = jnp.sum((xb != 0).astype(jnp.float32), axis=1, keepdims=True)
    h = jnp.maximum(sums_ref[...] / lengths + b1_ref[...], 0.0)
    out_ref[...] = (jnp.dot(h, w2_ref[...], preferred_element_type=jnp.float32)
                    + b2_ref[...])


def _mlp(x2d, sums, b1p, w2p, b2):
    return pl.pallas_call(
        _mlp_body,
        grid=(B // BLK,),
        in_specs=[
            pl.BlockSpec((BLK, L), lambda i: (i, 0)),
            pl.BlockSpec((BLK, HID2), lambda i: (i, 0)),
            pl.BlockSpec((1, HID2), lambda i: (0, 0)),
            pl.BlockSpec((HID2, OUT), lambda i: (0, 0)),
            pl.BlockSpec((1, OUT), lambda i: (0, 0)),
        ],
        out_specs=pl.BlockSpec((BLK, OUT), lambda i: (i, 0)),
        out_shape=jax.ShapeDtypeStruct((B, OUT), jnp.float32),
    )(x2d, sums, b1p.reshape(1, HID2), w2p, b2.reshape(1, OUT))


def kernel(x, table, W1, b1, W2, b2):
    x32 = x.astype(jnp.int32)
    w1p = jnp.zeros((DIM, HID2), jnp.float32).at[:, :HID].set(W1)
    b1p = jnp.zeros((HID2,), jnp.float32).at[:HID].set(b1)[_PERM]
    w2p = jnp.zeros((HID2, OUT), jnp.float32).at[:HID, :].set(W2)[_PERM, :]
    sums = _sc_sum()(x32, _pack_table(table, w1p))
    return _mlp(x32, sums, b1p, w2p, b2)
